# baseline (speedup 1.0000x reference)
import functools

import jax
import jax.numpy as jnp
from jax import lax
from jax.experimental import pallas as pl
from jax.experimental.pallas import tpu as pltpu
from jax.experimental.pallas import tpu_sc as plsc

NUM_CORES = 2
NUM_SUBCORES = 16
NUM_WORKERS = NUM_CORES * NUM_SUBCORES
LANES = 16


def _sc_body(x_hbm, p_hbm, o_hbm, p_v, x_v):
    B = x_hbm.shape[0]
    rows = p_v.shape[0]
    vecs_per_row = p_v.shape[1] // LANES
    wid = lax.axis_index("s") * NUM_CORES + lax.axis_index("c")
    base = wid * rows
    pltpu.sync_copy(p_hbm.at[pl.ds(base, rows)], p_v)
    for b in range(B):
        pltpu.sync_copy(x_hbm.at[b, pl.ds(base, rows)], x_v)

        def add_row(r, carry):
            for c in range(vecs_per_row):
                sl = (r, pl.ds(c * LANES, LANES))
                x_v[sl] = x_v[sl] + p_v[sl]
            return carry

        lax.fori_loop(0, rows, add_row, 0)
        pltpu.sync_copy(x_v, o_hbm.at[b, pl.ds(base, rows)])


def kernel(x, pos_table):
    B, T, D = x.shape
    rows = T // NUM_WORKERS
    mesh = plsc.VectorSubcoreMesh(core_axis_name="c", subcore_axis_name="s")
    run = functools.partial(
        pl.kernel,
        mesh=mesh,
        out_type=jax.ShapeDtypeStruct((B, T, D), jnp.float32),
        scratch_types=[
            pltpu.VMEM((rows, D), jnp.float32),
            pltpu.VMEM((rows, D), jnp.float32),
        ],
    )(_sc_body)
    return run(x, pos_table)


# speedup vs baseline: 1.2189x; 1.2189x over previous
import functools

import jax
import jax.numpy as jnp
from jax import lax
from jax.experimental import pallas as pl
from jax.experimental.pallas import tpu as pltpu
from jax.experimental.pallas import tpu_sc as plsc

NUM_CORES = 2
NUM_SUBCORES = 16
NUM_WORKERS = NUM_CORES * NUM_SUBCORES
LANES = 16
NBUF = 4
SUBCHUNKS = 4


def _sc_body(x_hbm, p_hbm, o_hbm, p_v, xbuf, obuf, lsem, ssem):
    B = x_hbm.shape[0]
    rows = p_v.shape[0]
    D = p_v.shape[1]
    vecs_per_row = D // LANES
    chunk = rows // SUBCHUNKS
    wid = lax.axis_index("s") * NUM_CORES + lax.axis_index("c")
    base = wid * rows
    pltpu.sync_copy(p_hbm.at[pl.ds(base, rows)], p_v)

    nchunks = B * SUBCHUNKS

    def load(g, k):
        b, s = g // SUBCHUNKS, g % SUBCHUNKS
        return pltpu.make_async_copy(
            x_hbm.at[b, pl.ds(base + s * chunk, chunk)], xbuf.at[k], lsem.at[k])

    def store(g, k):
        b, s = g // SUBCHUNKS, g % SUBCHUNKS
        return pltpu.make_async_copy(
            obuf.at[k], o_hbm.at[b, pl.ds(base + s * chunk, chunk)], ssem.at[k])

    for k in range(NBUF):
        load(k, k).start()

    for g in range(nchunks):
        k = g % NBUF
        s = g % SUBCHUNKS
        load(g, k).wait()
        if g >= NBUF:
            store(g - NBUF, k).wait()

        def add_row(r, carry):
            for c in range(vecs_per_row):
                sl = pl.ds(c * LANES, LANES)
                obuf[k, r, sl] = xbuf[k, r, sl] + p_v[s * chunk + r, sl]
            return carry

        lax.fori_loop(0, chunk, add_row, 0)
        store(g, k).start()
        if g + NBUF < nchunks:
            load(g + NBUF, k).start()

    for g in range(nchunks - NBUF, nchunks):
        store(g, g % NBUF).wait()


def kernel(x, pos_table):
    B, T, D = x.shape
    rows = T // NUM_WORKERS
    chunk = rows // SUBCHUNKS
    mesh = plsc.VectorSubcoreMesh(core_axis_name="c", subcore_axis_name="s")
    run = functools.partial(
        pl.kernel,
        mesh=mesh,
        out_type=jax.ShapeDtypeStruct((B, T, D), jnp.float32),
        scratch_types=[
            pltpu.VMEM((rows, D), jnp.float32),
            pltpu.VMEM((NBUF, chunk, D), jnp.float32),
            pltpu.VMEM((NBUF, chunk, D), jnp.float32),
            pltpu.SemaphoreType.DMA((NBUF,)),
            pltpu.SemaphoreType.DMA((NBUF,)),
        ],
    )(_sc_body)
    return run(x, pos_table)
